# Initial kernel scaffold; baseline (speedup 1.0000x reference)
import jax
import jax.numpy as jnp
from jax.experimental import pallas as pl


def kernel(x, C):
    raise NotImplementedError("write your pallas kernel here")


# speedup vs baseline: 1.4129x; 1.4129x over previous
import functools

import jax
import jax.numpy as jnp
from jax import lax
from jax.experimental import pallas as pl
from jax.experimental.pallas import tpu as pltpu
from jax.experimental.pallas import tpu_sc as plsc

N = 131072
D = 32
K = 512


R = 1024
NB = N // R


NC = 2
NS = 16
NW = NC * NS
RPW = N // NW
RPC = 512
NCH = RPW // RPC
GPC = RPC // 16
CW = RPC * D


def _dist_argmin_kernel(x_ref, c_ref, ids_ref):
    x = x_ref[...]
    c = c_ref[...]
    cnorm = jnp.sum(c * c, axis=0, keepdims=True)
    xnorm = jnp.sum(x * x, axis=1, keepdims=True)
    xc = jnp.dot(x, c, preferred_element_type=jnp.float32)
    dist = xnorm - 2.0 * xc + cnorm
    m = jnp.min(dist, axis=1, keepdims=True)
    iota = lax.broadcasted_iota(jnp.int32, (R, K), 1)
    ids = jnp.min(jnp.where(dist == m, iota, K), axis=1)
    ids_ref[0, 0, :] = ids


_dist_argmin = pl.pallas_call(
    _dist_argmin_kernel,
    grid=(NB,),
    in_specs=[
        pl.BlockSpec((R, D), lambda i: (i, 0)),
        pl.BlockSpec((D, K), lambda i: (0, 0)),
    ],
    out_specs=pl.BlockSpec((1, 1, R), lambda i: (i, 0, 0)),
    out_shape=jax.ShapeDtypeStruct((NB, 1, R), jnp.int32),
)


_sc_mesh = plsc.VectorSubcoreMesh(core_axis_name="c", subcore_axis_name="s")


@functools.partial(
    pl.kernel,
    mesh=_sc_mesh,
    out_type=jax.ShapeDtypeStruct((N * D,), jnp.float32),
    scratch_types=[
        pltpu.VMEM((K * D,), jnp.float32),
        pltpu.VMEM((RPW,), jnp.int32),
        pltpu.VMEM((CW,), jnp.float32),
    ],
    compiler_params=pltpu.CompilerParams(needs_layout_passes=False),
)
def _gather_codewords(table_hbm, idx_hbm, out_hbm, table_v, idx_v, out_c):
    wid = lax.axis_index("s") * NC + lax.axis_index("c")
    pltpu.sync_copy(table_hbm, table_v)
    pltpu.sync_copy(idx_hbm.at[pl.ds(wid * RPW, RPW)], idx_v)
    pos0 = lax.iota(jnp.int32, 16) * D

    def chunk_body(k, carry):
        def group_body(g, carry2):
            ids16 = idx_v[pl.ds(k * RPC + g * 16, 16)]
            base = ids16 * D
            pos = g * (16 * D) + pos0
            for col in range(D):
                v = plsc.load_gather(table_v, [base + col])
                plsc.store_scatter(out_c, [pos + col], v)
            return carry2

        lax.fori_loop(0, GPC, group_body, 0)
        pltpu.sync_copy(out_c, out_hbm.at[pl.ds((wid * NCH + k) * CW, CW)])
        return carry

    lax.fori_loop(0, NCH, chunk_body, 0)


def kernel(x, C):
    ids = _dist_argmin(x, C).reshape(N)
    table = C.T.reshape(K * D)
    out = _gather_codewords(table, ids)
    return out.reshape(N, D)


# speedup vs baseline: 1.6873x; 1.1943x over previous
import functools

import jax
import jax.numpy as jnp
from jax import lax
from jax.experimental import pallas as pl
from jax.experimental.pallas import tpu as pltpu
from jax.experimental.pallas import tpu_sc as plsc

N = 131072
D = 32
K = 512


R = 1024
NB = N // R


NC = 2
NS = 16
NW = NC * NS
RPW = N // NW
RPC = 512
NCH = RPW // RPC
GPC = RPC // 16
CW = RPC * D


def _dist_argmin_kernel(x_ref, c_ref, ids_ref):
    x = x_ref[...]
    c = c_ref[...]
    cnorm = jnp.sum(c * c, axis=0, keepdims=True)
    xnorm = jnp.sum(x * x, axis=1, keepdims=True)
    xc = jnp.dot(x, c, preferred_element_type=jnp.float32)
    dist = xnorm - 2.0 * xc + cnorm
    m = jnp.min(dist, axis=1, keepdims=True)
    iota = lax.broadcasted_iota(jnp.int32, (R, K), 1)
    ids = jnp.min(jnp.where(dist == m, iota, K), axis=1)
    ids_ref[0, 0, :] = ids


_dist_argmin = pl.pallas_call(
    _dist_argmin_kernel,
    grid=(NB,),
    in_specs=[
        pl.BlockSpec((R, D), lambda i: (i, 0)),
        pl.BlockSpec((D, K), lambda i: (0, 0)),
    ],
    out_specs=pl.BlockSpec((1, 1, R), lambda i: (i, 0, 0)),
    out_shape=jax.ShapeDtypeStruct((NB, 1, R), jnp.int32),
)


_sc_mesh = plsc.VectorSubcoreMesh(core_axis_name="c", subcore_axis_name="s")


@functools.partial(
    pl.kernel,
    mesh=_sc_mesh,
    out_type=jax.ShapeDtypeStruct((N * D,), jnp.float32),
    scratch_types=[
        pltpu.VMEM((K * D,), jnp.float32),
        pltpu.VMEM((RPW,), jnp.int32),
        pltpu.VMEM((CW,), jnp.float32),
    ],
    compiler_params=pltpu.CompilerParams(needs_layout_passes=False),
)
def _gather_codewords(table_hbm, idx_hbm, out_hbm, table_v, idx_v, out_c):
    wid = lax.axis_index("s") * NC + lax.axis_index("c")
    pltpu.sync_copy(table_hbm, table_v)
    pltpu.sync_copy(idx_hbm.at[pl.ds(wid * RPW, RPW)], idx_v)
    pos0 = lax.iota(jnp.int32, 16) * D

    def chunk_body(k, carry):
        @plsc.parallel_loop(0, GPC, unroll=2)
        def group_body(g):
            ids16 = idx_v[pl.ds(k * RPC + g * 16, 16)]
            pos = g * (16 * D) + pos0
            for col in range(D):
                v = plsc.load_gather(table_v, [ids16 + (col * K)])
                plsc.store_scatter(out_c, [pos + col], v)

        pltpu.sync_copy(out_c, out_hbm.at[pl.ds((wid * NCH + k) * CW, CW)])
        return carry

    lax.fori_loop(0, NCH, chunk_body, 0)


def kernel(x, C):
    ids = _dist_argmin(x, C).reshape(N)
    out = _gather_codewords(C.reshape(D * K), ids)
    return out.reshape(N, D)


# speedup vs baseline: 1.7007x; 1.0079x over previous
import functools

import jax
import jax.numpy as jnp
from jax import lax
from jax.experimental import pallas as pl
from jax.experimental.pallas import tpu as pltpu
from jax.experimental.pallas import tpu_sc as plsc

N = 131072
D = 32
K = 512


R = 1024
NB = N // R


NC = 2
NS = 16
NW = NC * NS
RPW = N // NW
RPC = 512
NCH = RPW // RPC
GPC = RPC // 16
CW = RPC * D


def _dist_argmin_kernel(x_ref, c_ref, ids_ref):
    x = x_ref[...]
    c = c_ref[...]
    cnorm = jnp.sum(c * c, axis=0, keepdims=True)
    xnorm = jnp.sum(x * x, axis=1, keepdims=True)
    xc = jnp.dot(x, c, preferred_element_type=jnp.float32)
    dist = xnorm - 2.0 * xc + cnorm
    m = jnp.min(dist, axis=1, keepdims=True)
    iota = lax.broadcasted_iota(jnp.int32, (R, K), 1)
    ids = jnp.min(jnp.where(dist == m, iota, K), axis=1)
    ids_ref[...] = ids


_dist_argmin = pl.pallas_call(
    _dist_argmin_kernel,
    grid=(NB,),
    in_specs=[
        pl.BlockSpec((R, D), lambda i: (i, 0)),
        pl.BlockSpec((D, K), lambda i: (0, 0)),
    ],
    out_specs=pl.BlockSpec((R,), lambda i: (i,)),
    out_shape=jax.ShapeDtypeStruct((N,), jnp.int32),
)


_sc_mesh = plsc.VectorSubcoreMesh(core_axis_name="c", subcore_axis_name="s")


@functools.partial(
    pl.kernel,
    mesh=_sc_mesh,
    out_type=jax.ShapeDtypeStruct((N * D,), jnp.float32),
    scratch_types=[
        pltpu.VMEM((D, K), jnp.float32),
        pltpu.VMEM((RPW,), jnp.int32),
        pltpu.VMEM((CW,), jnp.float32),
    ],
    compiler_params=pltpu.CompilerParams(needs_layout_passes=False),
)
def _gather_codewords(table_hbm, idx_hbm, out_hbm, table_v, idx_v, out_c):
    wid = lax.axis_index("s") * NC + lax.axis_index("c")
    pltpu.sync_copy(table_hbm, table_v)
    pltpu.sync_copy(idx_hbm.at[pl.ds(wid * RPW, RPW)], idx_v)
    pos0 = lax.iota(jnp.int32, 16) * D

    def chunk_body(k, carry):
        @plsc.parallel_loop(0, GPC, unroll=2)
        def group_body(g):
            ids16 = idx_v[pl.ds(k * RPC + g * 16, 16)]
            pos = g * (16 * D) + pos0
            for col in range(D):
                col_vec = jnp.full((16,), col, jnp.int32)
                v = plsc.load_gather(table_v, [col_vec, ids16])
                plsc.store_scatter(out_c, [pos + col], v)

        pltpu.sync_copy(out_c, out_hbm.at[pl.ds((wid * NCH + k) * CW, CW)])
        return carry

    lax.fori_loop(0, NCH, chunk_body, 0)


def kernel(x, C):
    ids = _dist_argmin(x, C)
    out = _gather_codewords(C, ids)
    return out.reshape(N, D)


# speedup vs baseline: 1.8319x; 1.0772x over previous
import functools

import jax
import jax.numpy as jnp
from jax import lax
from jax.experimental import pallas as pl
from jax.experimental.pallas import tpu as pltpu
from jax.experimental.pallas import tpu_sc as plsc

N = 131072
D = 32
K = 512


R = 1024
NB = N // R


NC = 2
NS = 16
NW = NC * NS
RPW = N // NW
RPC = 512
NCH = RPW // RPC
GPC = RPC // 16
CW = RPC * D


def _dist_argmin_kernel(x_ref, c_ref, ids_ref):
    x = x_ref[...]
    c = c_ref[...]
    cnorm = jnp.sum(c * c, axis=0, keepdims=True)
    xa = jnp.concatenate([x, jnp.ones((R, 1), jnp.float32)], axis=1)
    ca = jnp.concatenate([-2.0 * c, cnorm], axis=0)
    dist = jnp.dot(xa, ca, preferred_element_type=jnp.float32)
    ids_ref[...] = jnp.argmin(dist, axis=1).astype(jnp.int32)


_dist_argmin = pl.pallas_call(
    _dist_argmin_kernel,
    grid=(NB,),
    in_specs=[
        pl.BlockSpec((R, D), lambda i: (i, 0)),
        pl.BlockSpec((D, K), lambda i: (0, 0)),
    ],
    out_specs=pl.BlockSpec((R,), lambda i: (i,)),
    out_shape=jax.ShapeDtypeStruct((N,), jnp.int32),
)


_sc_mesh = plsc.VectorSubcoreMesh(core_axis_name="c", subcore_axis_name="s")


@functools.partial(
    pl.kernel,
    mesh=_sc_mesh,
    out_type=jax.ShapeDtypeStruct((N * D,), jnp.float32),
    scratch_types=[
        pltpu.VMEM((D, K), jnp.float32),
        pltpu.VMEM((RPW,), jnp.int32),
        pltpu.VMEM((CW,), jnp.float32),
    ],
    compiler_params=pltpu.CompilerParams(needs_layout_passes=False),
)
def _gather_codewords(table_hbm, idx_hbm, out_hbm, table_v, idx_v, out_c):
    wid = lax.axis_index("s") * NC + lax.axis_index("c")
    pltpu.sync_copy(table_hbm, table_v)
    pltpu.sync_copy(idx_hbm.at[pl.ds(wid * RPW, RPW)], idx_v)
    pos0 = lax.iota(jnp.int32, 16) * D

    def chunk_body(k, carry):
        @plsc.parallel_loop(0, GPC, unroll=2)
        def group_body(g):
            ids16 = idx_v[pl.ds(k * RPC + g * 16, 16)]
            pos = g * (16 * D) + pos0
            for col in range(D):
                col_vec = jnp.full((16,), col, jnp.int32)
                v = plsc.load_gather(table_v, [col_vec, ids16])
                plsc.store_scatter(out_c, [pos + col], v)

        pltpu.sync_copy(out_c, out_hbm.at[pl.ds((wid * NCH + k) * CW, CW)])
        return carry

    lax.fori_loop(0, NCH, chunk_body, 0)


def kernel(x, C):
    ids = _dist_argmin(x, C)
    out = _gather_codewords(C, ids)
    return out.reshape(N, D)
